# baseline (speedup 1.0000x reference)
import jax
import jax.numpy as jnp
from jax.experimental import pallas as pl
from jax.experimental.pallas import tpu as pltpu

B, T, D = 1, 2048, 1024
H, HD, NH = 16, 64, 2
L = T * NH
EPS = 1e-5
SCALE = HD ** -0.5
HHD = H * HD

PCOLS = 7168
PR_BM = 512

CT = 32
CHUNK = NH * CT
NC = T // CT

OB_M = 512


def _dot(a, b):
    return jax.lax.dot_general(a, b, (((1,), (0,)), ((), ())),
                               preferred_element_type=jnp.float32)


def _dot_nt(a, b):
    return jax.lax.dot_general(a, b, (((1,), (1,)), ((), ())),
                               preferred_element_type=jnp.float32)


def _dot_tn(a, b):
    return jax.lax.dot_general(a, b, (((0,), (0,)), ((), ())),
                               preferred_element_type=jnp.float32)


def _proj_body(x_ref, wq_ref, wk_ref, wv_ref, wg_ref, wba_ref, o_ref):
    j = pl.program_id(0)

    @pl.when(j == 0)
    def _():
        o_ref[...] = _dot(x_ref[...], wq_ref[...])

    @pl.when((j == 1) | (j == 2))
    def _():
        o_ref[...] = _dot(x_ref[...], wk_ref[...])

    @pl.when((j == 3) | (j == 4))
    def _():
        o_ref[...] = _dot(x_ref[...], wv_ref[...])

    @pl.when(j == 5)
    def _():
        o_ref[...] = _dot(x_ref[...], wg_ref[...])

    @pl.when(j == 6)
    def _():
        o_ref[:, 0:128] = _dot(x_ref[...], wba_ref[...])


def _delta_body(k_ref, v_ref, q_ref, b_ref, g_ref, gt_ref, o_ref, s_ref):
    c = pl.program_id(0)

    @pl.when(c == 0)
    def _():
        s_ref[...] = jnp.zeros_like(s_ref)

    C = CHUNK
    row = jax.lax.broadcasted_iota(jnp.int32, (C, C), 0)
    col = jax.lax.broadcasted_iota(jnp.int32, (C, C), 1)
    incl = row >= col
    strict = row > col
    lec = row <= col
    rtok = jax.lax.broadcasted_iota(jnp.int32, (CT, C), 0)
    codd = jax.lax.broadcasted_iota(jnp.int32, (CT, C), 1)
    incl_odd = codd <= 2 * rtok + 1

    kblk = k_ref[...]
    vblk = v_ref[...]
    qblk = q_ref[...]
    bblk = b_ref[...]
    gblk = g_ref[...]
    gtb = gt_ref[:, 0, 0]

    R = range(H)
    kn, qn, v, bcol, eG, eGlast, eCI, Dincl, Dstrict, S, eGo = \
        [], [], [], [], [], [], [], [], [], [], []
    for i in R:
        k = kblk[:, i * HD:(i + 1) * HD]
        q = qblk[:, i * HD:(i + 1) * HD]
        v.append(vblk[:, i * HD:(i + 1) * HD])
        bcol.append(bblk[:, i:i + 1])
        gcol = gblk[:, i:i + 1]
        grow = gtb[i:i + 1, :]
        S.append(s_ref[i])

        kn.append(k * jax.lax.rsqrt(jnp.sum(k * k, axis=1, keepdims=True) + 1e-6))
        qn.append(q * jax.lax.rsqrt(jnp.sum(q * q, axis=1, keepdims=True) + 1e-6)
                  * SCALE)

        Grow = jnp.sum(jnp.where(incl, jnp.broadcast_to(grow, (C, C)), 0.0),
                       axis=1, keepdims=True)
        Gcol = jnp.sum(jnp.where(lec, jnp.broadcast_to(gcol, (C, C)), 0.0),
                       axis=0, keepdims=True)
        Godd = jnp.sum(jnp.where(incl_odd, jnp.broadcast_to(grow, (CT, C)), 0.0),
                       axis=1, keepdims=True)
        eG.append(jnp.exp(Grow))
        eGo.append(jnp.exp(Godd))
        Glast = jnp.sum(grow)
        eGlast.append(jnp.exp(Glast))
        eCI.append(jnp.exp(Glast - Grow))
        Dfull = jnp.exp(jnp.where(incl, Grow - Gcol, -1e30))
        Dstrict.append(jnp.where(strict, Dfull, 0.0))
        Dodd = jnp.exp(jnp.where(incl_odd, Godd - Gcol, -1e30))
        Dincl.append(jnp.where(incl_odd, Dodd, 0.0))

    Np = [_dot_nt(kn[i] * bcol[i], kn[i]) * (-Dstrict[i]) for i in R]
    pred = [_dot(kn[i], S[i]) for i in R]
    attn = [_dot_nt(qn[i], kn[i]) * Dincl[i] for i in R]
    oq = [_dot(qn[i], S[i]) for i in R]
    t = [bcol[i] * (v[i] - eG[i] * pred[i]) for i in R]

    for j in range(6):
        t = [t[i] + _dot(Np[i], t[i]) for i in R]
        if j < 5:
            Np = [_dot(Np[i], Np[i]) for i in R]

    o_ref[...] = jnp.concatenate(
        [eGo[i] * oq[i] + _dot(attn[i], t[i]) for i in R], axis=1)
    for i in R:
        s_ref[i] = eGlast[i] * S[i] + _dot_tn(kn[i] * eCI[i], t[i])


def _out_body(o_ref, g_ref, w_ref, nw_ref, y_ref):
    o = o_ref[...]
    lane = jax.lax.broadcasted_iota(jnp.int32, (HHD, H), 0)
    grp = jax.lax.broadcasted_iota(jnp.int32, (HHD, H), 1)
    ind = (lane // HD == grp).astype(jnp.float32)
    ms = _dot(o * o, ind) * (1.0 / HD)
    rexp = _dot(jax.lax.rsqrt(ms + EPS), ind.T)
    gg = g_ref[...]
    y = o * rexp * nw_ref[...] * (gg * jax.nn.sigmoid(gg))
    y_ref[...] = _dot(y, w_ref[...])


def kernel(x, Wq, Wk, Wv, Wb, Wa, A_log, dt_bias, Wg, norm_weight, Wo):
    x2 = x.reshape(T, D)
    Wba = jnp.pad(jnp.concatenate([Wb, Wa], axis=1), ((0, 0), (0, 80)))

    _c0 = lambda j, i: (0, 0)
    P = pl.pallas_call(
        _proj_body,
        out_shape=jax.ShapeDtypeStruct((T, PCOLS), jnp.float32),
        grid=(7, T // PR_BM),
        in_specs=[
            pl.BlockSpec((PR_BM, D), lambda j, i: (i, 0)),
            pl.BlockSpec((D, 1024), _c0),
            pl.BlockSpec((D, 1024),
                         lambda j, i: (0, jnp.clip(j - 1, 0, 1))),
            pl.BlockSpec((D, 1024),
                         lambda j, i: (0, jnp.clip(j - 3, 0, 1))),
            pl.BlockSpec((D, 1024), _c0),
            pl.BlockSpec((D, 128), _c0),
        ],
        out_specs=pl.BlockSpec((PR_BM, 1024), lambda j, i: (i, j)),
        compiler_params=pltpu.CompilerParams(
            dimension_semantics=("arbitrary", "arbitrary"),
            vmem_limit_bytes=52 * 1024 * 1024),
        name="deltanet_proj",
    )(x2, Wq, Wk, Wv, Wg, Wba)

    karr = P[:, 1024:3072].reshape(L, HHD)
    varr = P[:, 3072:5120].reshape(L, HHD)
    braw = P[:, 6144:6176]
    araw = P[:, 6176:6192]

    barr = (2.0 * jax.nn.sigmoid(braw)).reshape(L, H)
    g_tok = -jnp.exp(A_log)[None, :] * jax.nn.softplus(araw + dt_bias[None, :])
    garr = jnp.stack([g_tok, jnp.zeros((T, H), jnp.float32)], axis=1).reshape(L, H)
    gtarr = garr.T.reshape(H, NC, 1, CHUNK)

    O2d = pl.pallas_call(
        _delta_body,
        out_shape=jax.ShapeDtypeStruct((T, HHD), jnp.float32),
        grid=(NC,),
        in_specs=[
            pl.BlockSpec((CHUNK, HHD), lambda c: (c, 0)),
            pl.BlockSpec((CHUNK, HHD), lambda c: (c, 0)),
            pl.BlockSpec((CT, 1024), lambda c: (c, 0)),
            pl.BlockSpec((CHUNK, H), lambda c: (c, 0)),
            pl.BlockSpec((CHUNK, H), lambda c: (c, 0)),
            pl.BlockSpec((H, 1, 1, CHUNK), lambda c: (0, c, 0, 0)),
        ],
        out_specs=pl.BlockSpec((CT, HHD), lambda c: (c, 0)),
        scratch_shapes=[pltpu.VMEM((H, HD, HD), jnp.float32)],
        compiler_params=pltpu.CompilerParams(
            dimension_semantics=("arbitrary",)),
        name="deltanet_chunk",
    )(karr, varr, P, barr, garr, gtarr)

    nw_full = jnp.tile(norm_weight, (H,)).reshape(1, HHD)
    y = pl.pallas_call(
        _out_body,
        out_shape=jax.ShapeDtypeStruct((T, D), jnp.float32),
        grid=(T // OB_M,),
        in_specs=[
            pl.BlockSpec((OB_M, HHD), lambda i: (i, 0)),
            pl.BlockSpec((OB_M, 1024), lambda i: (i, 5)),
            pl.BlockSpec((HHD, D), lambda i: (0, 0)),
            pl.BlockSpec((1, HHD), lambda i: (0, 0)),
        ],
        out_specs=pl.BlockSpec((OB_M, D), lambda i: (i, 0)),
        compiler_params=pltpu.CompilerParams(
            dimension_semantics=("arbitrary",)),
        name="deltanet_out",
    )(O2d, P, Wo, nw_full)

    return y.reshape(B, T, D)


# speedup vs baseline: 1.3305x; 1.3305x over previous
import jax
import jax.numpy as jnp
from jax.experimental import pallas as pl
from jax.experimental.pallas import tpu as pltpu

B, T, D = 1, 2048, 1024
H, HD, NH = 16, 64, 2
L = T * NH
EPS = 1e-5
SCALE = HD ** -0.5

PCOLS = 7168
PR_BM = 512

CT = 32
CHUNK = NH * CT
NC = T // CT

OB_M = 512

GO = 5120


def _dot(a, b):
    return jax.lax.dot_general(a, b, (((1,), (0,)), ((), ())),
                               preferred_element_type=jnp.float32)


def _dot_nt(a, b):
    return jax.lax.dot_general(a, b, (((1,), (1,)), ((), ())),
                               preferred_element_type=jnp.float32)


def _dot_tn(a, b):
    return jax.lax.dot_general(a, b, (((0,), (0,)), ((), ())),
                               preferred_element_type=jnp.float32)


def _proj_body(x_ref, wq_ref, wk_ref, wv_ref, wg_ref, wba_ref, o_ref):
    j = pl.program_id(0)

    @pl.when(j == 0)
    def _():
        o_ref[...] = _dot(x_ref[...], wq_ref[...])

    @pl.when((j == 1) | (j == 2))
    def _():
        o_ref[...] = _dot(x_ref[...], wk_ref[...])

    @pl.when((j == 3) | (j == 4))
    def _():
        o_ref[...] = _dot(x_ref[...], wv_ref[...])

    @pl.when(j == 5)
    def _():
        o_ref[...] = _dot(x_ref[...], wg_ref[...])

    @pl.when(j == 6)
    def _():
        o_ref[:, 0:128] = _dot(x_ref[...], wba_ref[...])


def _delta_body(k_ref, v_ref, q_ref, bc_ref, gc_ref, gr_ref, o_ref, s_ref):
    c = pl.program_id(0)

    @pl.when(c == 0)
    def _():
        s_ref[...] = jnp.zeros_like(s_ref)

    C = CHUNK
    row = jax.lax.broadcasted_iota(jnp.int32, (C, C), 0)
    col = jax.lax.broadcasted_iota(jnp.int32, (C, C), 1)
    incl = row >= col
    strict = row > col
    lec = row <= col
    rtok = jax.lax.broadcasted_iota(jnp.int32, (CT, C), 0)
    codd = jax.lax.broadcasted_iota(jnp.int32, (CT, C), 1)
    incl_odd = codd <= 2 * rtok + 1

    R = range(H)
    kn, qn, v, bcol, eG, eGlast, eCI, Dincl, Dstrict, S, eGo = \
        [], [], [], [], [], [], [], [], [], [], []
    for i in R:
        k = k_ref[i]
        q = q_ref[i]
        v.append(v_ref[i])
        bcol.append(bc_ref[0][:, i:i + 1])
        gcol = gc_ref[0][:, i:i + 1]
        grow = gr_ref[i, 0]
        S.append(s_ref[i])

        kn.append(k * jax.lax.rsqrt(jnp.sum(k * k, axis=1, keepdims=True) + 1e-6))
        qn.append(q * jax.lax.rsqrt(jnp.sum(q * q, axis=1, keepdims=True) + 1e-6)
                  * SCALE)

        Grow = jnp.sum(jnp.where(incl, jnp.broadcast_to(grow, (C, C)), 0.0),
                       axis=1, keepdims=True)
        Gcol = jnp.sum(jnp.where(lec, jnp.broadcast_to(gcol, (C, C)), 0.0),
                       axis=0, keepdims=True)
        Godd = jnp.sum(jnp.where(incl_odd, jnp.broadcast_to(grow, (CT, C)), 0.0),
                       axis=1, keepdims=True)
        eG.append(jnp.exp(Grow))
        eGo.append(jnp.exp(Godd))
        Glast = jnp.sum(grow)
        eGlast.append(jnp.exp(Glast))
        eCI.append(jnp.exp(Glast - Grow))
        Dfull = jnp.exp(jnp.where(incl, Grow - Gcol, -1e30))
        Dstrict.append(jnp.where(strict, Dfull, 0.0))
        Dodd = jnp.exp(jnp.where(incl_odd, Godd - Gcol, -1e30))
        Dincl.append(jnp.where(incl_odd, Dodd, 0.0))

    Np = [_dot_nt(kn[i] * bcol[i], kn[i]) * (-Dstrict[i]) for i in R]
    pred = [_dot(kn[i], S[i]) for i in R]
    attn = [_dot_nt(qn[i], kn[i]) * Dincl[i] for i in R]
    oq = [_dot(qn[i], S[i]) for i in R]
    t = [bcol[i] * (v[i] - eG[i] * pred[i]) for i in R]

    for j in range(6):
        t = [t[i] + _dot(Np[i], t[i]) for i in R]
        if j < 5:
            Np = [_dot(Np[i], Np[i]) for i in R]

    for i in R:
        o_ref[:, i * HD:(i + 1) * HD] = eGo[i] * oq[i] + _dot(attn[i], t[i])
    for i in R:
        s_ref[i] = eGlast[i] * S[i] + _dot_tn(kn[i] * eCI[i], t[i])


def _out_body(o_ref, g_ref, w_ref, nw_ref, y_ref):
    o = o_ref[...]
    lane = jax.lax.broadcasted_iota(jnp.int32, (H * HD, H), 0)
    grp = jax.lax.broadcasted_iota(jnp.int32, (H * HD, H), 1)
    ind = (lane // HD == grp).astype(jnp.float32)
    ms = _dot(o * o, ind) * (1.0 / HD)
    rexp = _dot(jax.lax.rsqrt(ms + EPS), ind.T)
    gg = g_ref[...]
    y = o * rexp * nw_ref[...] * (gg * jax.nn.sigmoid(gg))
    y_ref[...] = _dot(y, w_ref[...])


def kernel(x, Wq, Wk, Wv, Wb, Wa, A_log, dt_bias, Wg, norm_weight, Wo):
    x2 = x.reshape(T, D)
    Wba = jnp.pad(jnp.concatenate([Wb, Wa], axis=1), ((0, 0), (0, 80)))

    _c0 = lambda j, i: (0, 0)
    P = pl.pallas_call(
        _proj_body,
        out_shape=jax.ShapeDtypeStruct((T, PCOLS), jnp.float32),
        grid=(7, T // PR_BM),
        in_specs=[
            pl.BlockSpec((PR_BM, D), lambda j, i: (i, 0)),
            pl.BlockSpec((D, 1024), _c0),
            pl.BlockSpec((D, 1024),
                         lambda j, i: (0, jnp.clip(j - 1, 0, 1))),
            pl.BlockSpec((D, 1024),
                         lambda j, i: (0, jnp.clip(j - 3, 0, 1))),
            pl.BlockSpec((D, 1024), _c0),
            pl.BlockSpec((D, 128), _c0),
        ],
        out_specs=pl.BlockSpec((PR_BM, 1024), lambda j, i: (i, j)),
        compiler_params=pltpu.CompilerParams(
            dimension_semantics=("arbitrary", "arbitrary"),
            vmem_limit_bytes=52 * 1024 * 1024),
        name="deltanet_proj",
    )(x2, Wq, Wk, Wv, Wg, Wba)

    qraw = P[:, 0:1024]
    kraw = P[:, 1024:3072]
    vraw = P[:, 3072:5120]
    braw = P[:, 6144:6176]
    araw = P[:, 6176:6192]

    qarr = qraw.reshape(T, H, HD).transpose(1, 0, 2)
    kstep = kraw.reshape(T, NH, H, HD).transpose(2, 0, 1, 3).reshape(H, L, HD)
    vstep = vraw.reshape(T, NH, H, HD).transpose(2, 0, 1, 3).reshape(H, L, HD)

    b_arr = (2.0 * jax.nn.sigmoid(braw)).reshape(NC, CHUNK, H)
    g_tok = -jnp.exp(A_log)[None, :] * jax.nn.softplus(araw + dt_bias[None, :])
    gstep = jnp.stack([g_tok, jnp.zeros((T, H), jnp.float32)], axis=1).reshape(L, H)
    g_arr = gstep.reshape(NC, CHUNK, H)
    g_row = gstep.T.reshape(H, NC, 1, CHUNK)

    O2d = pl.pallas_call(
        _delta_body,
        out_shape=jax.ShapeDtypeStruct((T, H * HD), jnp.float32),
        grid=(NC,),
        in_specs=[
            pl.BlockSpec((H, CHUNK, HD), lambda c: (0, c, 0)),
            pl.BlockSpec((H, CHUNK, HD), lambda c: (0, c, 0)),
            pl.BlockSpec((H, CT, HD), lambda c: (0, c, 0)),
            pl.BlockSpec((1, CHUNK, H), lambda c: (c, 0, 0)),
            pl.BlockSpec((1, CHUNK, H), lambda c: (c, 0, 0)),
            pl.BlockSpec((H, 1, 1, CHUNK), lambda c: (0, c, 0, 0)),
        ],
        out_specs=pl.BlockSpec((CT, H * HD), lambda c: (c, 0)),
        scratch_shapes=[pltpu.VMEM((H, HD, HD), jnp.float32)],
        compiler_params=pltpu.CompilerParams(
            dimension_semantics=("arbitrary",)),
        name="deltanet_chunk",
    )(kstep, vstep, qarr, b_arr, g_arr, g_row)

    nw_full = jnp.tile(norm_weight, (H,)).reshape(1, H * HD)
    y = pl.pallas_call(
        _out_body,
        out_shape=jax.ShapeDtypeStruct((T, D), jnp.float32),
        grid=(T // OB_M,),
        in_specs=[
            pl.BlockSpec((OB_M, H * HD), lambda i: (i, 0)),
            pl.BlockSpec((OB_M, 1024), lambda i: (i, 5)),
            pl.BlockSpec((H * HD, D), lambda i: (0, 0)),
            pl.BlockSpec((1, H * HD), lambda i: (0, 0)),
        ],
        out_specs=pl.BlockSpec((OB_M, D), lambda i: (i, 0)),
        compiler_params=pltpu.CompilerParams(
            dimension_semantics=("arbitrary",)),
        name="deltanet_out",
    )(O2d, P, Wo, nw_full)

    return y.reshape(B, T, D)


# speedup vs baseline: 1.3490x; 1.0139x over previous
import jax
import jax.numpy as jnp
from jax.experimental import pallas as pl
from jax.experimental.pallas import tpu as pltpu

B, T, D = 1, 2048, 1024
H, HD, NH = 16, 64, 2
L = T * NH
EPS = 1e-5
SCALE = HD ** -0.5

PCOLS = 7168
PR_BM = 1024

CT = 32
CHUNK = NH * CT
NC = T // CT

OB_M = 512

GO = 5120


def _dot(a, b):
    return jax.lax.dot_general(a, b, (((1,), (0,)), ((), ())),
                               preferred_element_type=jnp.float32)


def _dot_nt(a, b):
    return jax.lax.dot_general(a, b, (((1,), (1,)), ((), ())),
                               preferred_element_type=jnp.float32)


def _dot_tn(a, b):
    return jax.lax.dot_general(a, b, (((0,), (0,)), ((), ())),
                               preferred_element_type=jnp.float32)


def _proj_body(x_ref, wq_ref, wk_ref, wv_ref, wg_ref, wba_ref, o_ref):
    j = pl.program_id(0)

    @pl.when(j == 0)
    def _():
        o_ref[...] = _dot(x_ref[...], wq_ref[...])

    @pl.when((j == 1) | (j == 2))
    def _():
        o_ref[...] = _dot(x_ref[...], wk_ref[...])

    @pl.when((j == 3) | (j == 4))
    def _():
        o_ref[...] = _dot(x_ref[...], wv_ref[...])

    @pl.when(j == 5)
    def _():
        o_ref[...] = _dot(x_ref[...], wg_ref[...])

    @pl.when(j == 6)
    def _():
        o_ref[:, 0:128] = _dot(x_ref[...], wba_ref[...])


def _delta_body(k_ref, v_ref, q_ref, bc_ref, gc_ref, gr_ref, o_ref, s_ref):
    c = pl.program_id(0)

    @pl.when(c == 0)
    def _():
        s_ref[...] = jnp.zeros_like(s_ref)

    C = CHUNK
    row = jax.lax.broadcasted_iota(jnp.int32, (C, C), 0)
    col = jax.lax.broadcasted_iota(jnp.int32, (C, C), 1)
    incl = row >= col
    strict = row > col
    lec = row <= col
    rtok = jax.lax.broadcasted_iota(jnp.int32, (CT, C), 0)
    codd = jax.lax.broadcasted_iota(jnp.int32, (CT, C), 1)
    incl_odd = codd <= 2 * rtok + 1

    R = range(H)
    kn, qn, v, bcol, eG, eGlast, eCI, Dincl, Dstrict, S, eGo = \
        [], [], [], [], [], [], [], [], [], [], []
    for i in R:
        k = k_ref[i]
        q = q_ref[i]
        v.append(v_ref[i])
        bcol.append(bc_ref[0][:, i:i + 1])
        gcol = gc_ref[0][:, i:i + 1]
        grow = gr_ref[i, 0]
        S.append(s_ref[i])

        kn.append(k * jax.lax.rsqrt(jnp.sum(k * k, axis=1, keepdims=True) + 1e-6))
        qn.append(q * jax.lax.rsqrt(jnp.sum(q * q, axis=1, keepdims=True) + 1e-6)
                  * SCALE)

        Grow = jnp.sum(jnp.where(incl, jnp.broadcast_to(grow, (C, C)), 0.0),
                       axis=1, keepdims=True)
        Gcol = jnp.sum(jnp.where(lec, jnp.broadcast_to(gcol, (C, C)), 0.0),
                       axis=0, keepdims=True)
        Godd = jnp.sum(jnp.where(incl_odd, jnp.broadcast_to(grow, (CT, C)), 0.0),
                       axis=1, keepdims=True)
        eG.append(jnp.exp(Grow))
        eGo.append(jnp.exp(Godd))
        Glast = jnp.sum(grow)
        eGlast.append(jnp.exp(Glast))
        eCI.append(jnp.exp(Glast - Grow))
        Dfull = jnp.exp(jnp.where(incl, Grow - Gcol, -1e30))
        Dstrict.append(jnp.where(strict, Dfull, 0.0))
        Dodd = jnp.exp(jnp.where(incl_odd, Godd - Gcol, -1e30))
        Dincl.append(jnp.where(incl_odd, Dodd, 0.0))

    Np = [_dot_nt(kn[i] * bcol[i], kn[i]) * (-Dstrict[i]) for i in R]
    pred = [_dot(kn[i], S[i]) for i in R]
    attn = [_dot_nt(qn[i], kn[i]) * Dincl[i] for i in R]
    oq = [_dot(qn[i], S[i]) for i in R]
    t = [bcol[i] * (v[i] - eG[i] * pred[i]) for i in R]

    for j in range(6):
        t = [t[i] + _dot(Np[i], t[i]) for i in R]
        if j < 5:
            Np = [_dot(Np[i], Np[i]) for i in R]

    for i in R:
        o_ref[:, i * HD:(i + 1) * HD] = eGo[i] * oq[i] + _dot(attn[i], t[i])
    for i in R:
        s_ref[i] = eGlast[i] * S[i] + _dot_tn(kn[i] * eCI[i], t[i])


def _out_body(o_ref, g_ref, w_ref, nw_ref, y_ref):
    o = o_ref[...]
    lane = jax.lax.broadcasted_iota(jnp.int32, (H * HD, H), 0)
    grp = jax.lax.broadcasted_iota(jnp.int32, (H * HD, H), 1)
    ind = (lane // HD == grp).astype(jnp.float32)
    ms = _dot(o * o, ind) * (1.0 / HD)
    rexp = _dot(jax.lax.rsqrt(ms + EPS), ind.T)
    gg = g_ref[...]
    y = o * rexp * nw_ref[...] * (gg * jax.nn.sigmoid(gg))
    y_ref[...] = _dot(y, w_ref[...])


def kernel(x, Wq, Wk, Wv, Wb, Wa, A_log, dt_bias, Wg, norm_weight, Wo):
    x2 = x.reshape(T, D)
    Wba = jnp.pad(jnp.concatenate([Wb, Wa], axis=1), ((0, 0), (0, 80)))

    _c0 = lambda j, i: (0, 0)
    P = pl.pallas_call(
        _proj_body,
        out_shape=jax.ShapeDtypeStruct((T, PCOLS), jnp.float32),
        grid=(7, T // PR_BM),
        in_specs=[
            pl.BlockSpec((PR_BM, D), lambda j, i: (i, 0)),
            pl.BlockSpec((D, 1024), _c0),
            pl.BlockSpec((D, 1024),
                         lambda j, i: (0, jnp.clip(j - 1, 0, 1))),
            pl.BlockSpec((D, 1024),
                         lambda j, i: (0, jnp.clip(j - 3, 0, 1))),
            pl.BlockSpec((D, 1024), _c0),
            pl.BlockSpec((D, 128), _c0),
        ],
        out_specs=pl.BlockSpec((PR_BM, 1024), lambda j, i: (i, j)),
        compiler_params=pltpu.CompilerParams(
            dimension_semantics=("arbitrary", "arbitrary"),
            vmem_limit_bytes=52 * 1024 * 1024),
        name="deltanet_proj",
    )(x2, Wq, Wk, Wv, Wg, Wba)

    qraw = P[:, 0:1024]
    kraw = P[:, 1024:3072]
    vraw = P[:, 3072:5120]
    braw = P[:, 6144:6176]
    araw = P[:, 6176:6192]

    qarr = qraw.reshape(T, H, HD).transpose(1, 0, 2)
    kstep = kraw.reshape(T, NH, H, HD).transpose(2, 0, 1, 3).reshape(H, L, HD)
    vstep = vraw.reshape(T, NH, H, HD).transpose(2, 0, 1, 3).reshape(H, L, HD)

    b_arr = (2.0 * jax.nn.sigmoid(braw)).reshape(NC, CHUNK, H)
    g_tok = -jnp.exp(A_log)[None, :] * jax.nn.softplus(araw + dt_bias[None, :])
    gstep = jnp.stack([g_tok, jnp.zeros((T, H), jnp.float32)], axis=1).reshape(L, H)
    g_arr = gstep.reshape(NC, CHUNK, H)
    g_row = gstep.T.reshape(H, NC, 1, CHUNK)

    O2d = pl.pallas_call(
        _delta_body,
        out_shape=jax.ShapeDtypeStruct((T, H * HD), jnp.float32),
        grid=(NC,),
        in_specs=[
            pl.BlockSpec((H, CHUNK, HD), lambda c: (0, c, 0)),
            pl.BlockSpec((H, CHUNK, HD), lambda c: (0, c, 0)),
            pl.BlockSpec((H, CT, HD), lambda c: (0, c, 0)),
            pl.BlockSpec((1, CHUNK, H), lambda c: (c, 0, 0)),
            pl.BlockSpec((1, CHUNK, H), lambda c: (c, 0, 0)),
            pl.BlockSpec((H, 1, 1, CHUNK), lambda c: (0, c, 0, 0)),
        ],
        out_specs=pl.BlockSpec((CT, H * HD), lambda c: (c, 0)),
        scratch_shapes=[pltpu.VMEM((H, HD, HD), jnp.float32)],
        compiler_params=pltpu.CompilerParams(
            dimension_semantics=("arbitrary",)),
        name="deltanet_chunk",
    )(kstep, vstep, qarr, b_arr, g_arr, g_row)

    nw_full = jnp.tile(norm_weight, (H,)).reshape(1, H * HD)
    y = pl.pallas_call(
        _out_body,
        out_shape=jax.ShapeDtypeStruct((T, D), jnp.float32),
        grid=(T // OB_M,),
        in_specs=[
            pl.BlockSpec((OB_M, H * HD), lambda i: (i, 0)),
            pl.BlockSpec((OB_M, 1024), lambda i: (i, 5)),
            pl.BlockSpec((H * HD, D), lambda i: (0, 0)),
            pl.BlockSpec((1, H * HD), lambda i: (0, 0)),
        ],
        out_specs=pl.BlockSpec((OB_M, D), lambda i: (i, 0)),
        compiler_params=pltpu.CompilerParams(
            dimension_semantics=("arbitrary",)),
        name="deltanet_out",
    )(O2d, P, Wo, nw_full)

    return y.reshape(B, T, D)
